# baseline (speedup 1.0000x reference)
import functools

import jax
import jax.numpy as jnp
from jax import lax
from jax.experimental import pallas as pl
from jax.experimental.pallas import tpu as pltpu
from jax.experimental.pallas import tpu_sc as plsc

_BINS = 50
_EPS = 1e-08

_NC = 2
_NS = 16
_L = 16

_BINS_PAD = 64
_NCOPY = 8
_HWORDS = _BINS_PAD * _L * _NCOPY
_SC_CHUNK = 32768


def _pass1_body(nchunk, p_ref, t_ref, o_ref, s_ref, mm_ref):
    j = pl.program_id(0)
    p = p_ref[...]
    t = t_ref[...]
    vp = jnp.maximum(jnp.sqrt(p[:, 0, :] * p[:, 0, :] + p[:, 1, :] * p[:, 1, :]), 1e-6)
    vt = jnp.maximum(jnp.sqrt(t[:, 0, :] * t[:, 0, :] + t[:, 1, :] * t[:, 1, :]), 1e-6)
    o_ref[0:16, :] = vp
    o_ref[16:32, :] = vt
    bmin = jnp.minimum(jnp.min(vp), jnp.min(vt))
    bmax = jnp.maximum(jnp.max(vp), jnp.max(vt))

    @pl.when(j == 0)
    def _():
        mm_ref[0] = bmin
        mm_ref[1] = bmax

    @pl.when(j > 0)
    def _():
        mm_ref[0] = jnp.minimum(mm_ref[0], bmin)
        mm_ref[1] = jnp.maximum(mm_ref[1], bmax)

    @pl.when(j == nchunk - 1)
    def _():
        mn = mm_ref[0]
        mx = mm_ref[1]
        scale = _BINS / jnp.maximum(mx - mn, 1e-12)
        s_ref[0, 0] = mn
        s_ref[0, 1] = scale


def _binpass_body(y_ref, s_ref, o_ref):
    mn = s_ref[0, 0]
    sc = s_ref[0, 1]
    y = y_ref[...]
    t = (y - mn) * sc
    t = jnp.maximum(jnp.minimum(t, float(_BINS - 1)), 0.0)
    idx = t.astype(jnp.int32)
    ci = lax.broadcasted_iota(jnp.int32, y.shape, 1)
    pat = (ci & (_L - 1)) + ((ci >> 4) & (_NCOPY - 1)) * (_BINS_PAD * _L)
    o_ref[...] = idx * _L + pat


def _sc_hist_body(n, fidx, out, buf0, buf1, hist, sem0, sem1):
    c = lax.axis_index("c")
    s = lax.axis_index("s")
    wid = c * _NS + s

    zv = jnp.zeros((_L,), jnp.float32)

    @plsc.parallel_loop(0, _HWORDS // _L, unroll=4)
    def _zero(i):
        hist[pl.ds(i * _L, _L)] = zv

    ones = jnp.ones((_L,), jnp.float32)
    nchunk = n // _SC_CHUNK
    nvec = _SC_CHUNK // _L
    bufs = (buf0, buf1)
    sems = (sem0, sem1)
    copies = [None, None]
    copies[0] = pltpu.async_copy(fidx.at[wid, pl.ds(0, _SC_CHUNK)], buf0, sem0)
    for k in range(nchunk):
        cur = k % 2
        if k + 1 < nchunk:
            copies[1 - cur] = pltpu.async_copy(
                fidx.at[wid, pl.ds((k + 1) * _SC_CHUNK, _SC_CHUNK)],
                bufs[1 - cur],
                sems[1 - cur],
            )
        copies[cur].wait()
        buf = bufs[cur]

        @plsc.parallel_loop(0, nvec, unroll=8)
        def body(i):
            fl = buf[pl.ds(i * _L, _L)]
            plsc.addupdate_scatter(hist, [fl], ones)

    pltpu.sync_copy(hist, out.at[wid])


def _pass2_body(h_ref, o_ref):
    xs = h_ref[...]
    x = xs[:, 0 : _BINS_PAD * _L]
    for j in range(1, _NCOPY):
        x = x + xs[:, j * _BINS_PAD * _L : (j + 1) * _BINS_PAD * _L]
    rows = lax.broadcasted_iota(jnp.int32, (_BINS_PAD * _L, 128), 0)
    cols = lax.broadcasted_iota(jnp.int32, (_BINS_PAD * _L, 128), 1)
    m = (rows // _L == cols).astype(jnp.float32)
    h = jnp.dot(x, m, preferred_element_type=jnp.float32)
    lanes = lax.broadcasted_iota(jnp.int32, (16, 128), 1)
    valid = lanes < _BINS
    hp = jnp.where(valid, h[0:16, :] + _EPS, 0.0)
    ht = jnp.where(valid, h[16:32, :] + _EPS, 0.0)
    hp = hp / jnp.sum(hp, axis=1, keepdims=True)
    ht = ht / jnp.sum(ht, axis=1, keepdims=True)
    kl = jnp.where(valid, ht * jnp.log(ht / hp), 0.0)
    o_ref[0, 0] = jnp.sum(kl) / 16.0


def kernel(y_pred, y_true):
    B, C, H, W = y_pred.shape
    N = H * W
    yp3 = y_pred.reshape(B, C, N)
    yt3 = y_true.reshape(B, C, N)

    ch = 4096
    nchunk = N // ch
    yext, stats = pl.pallas_call(
        functools.partial(_pass1_body, nchunk),
        grid=(nchunk,),
        in_specs=[
            pl.BlockSpec((B, C, ch), lambda j: (0, 0, j)),
            pl.BlockSpec((B, C, ch), lambda j: (0, 0, j)),
        ],
        out_specs=[
            pl.BlockSpec((2 * B, ch), lambda j: (0, j)),
            pl.BlockSpec((1, 2), lambda j: (0, 0), memory_space=pltpu.SMEM),
        ],
        out_shape=[
            jax.ShapeDtypeStruct((2 * B, N), jnp.float32),
            jax.ShapeDtypeStruct((1, 2), jnp.float32),
        ],
        scratch_shapes=[pltpu.SMEM((2,), jnp.float32)],
    )(yp3, yt3)

    fidx = pl.pallas_call(
        _binpass_body,
        grid=(nchunk,),
        in_specs=[
            pl.BlockSpec((2 * B, ch), lambda j: (0, j)),
            pl.BlockSpec((1, 2), lambda j: (0, 0), memory_space=pltpu.SMEM),
        ],
        out_specs=pl.BlockSpec((2 * B, ch), lambda j: (0, j)),
        out_shape=jax.ShapeDtypeStruct((2 * B, N), jnp.int32),
    )(yext, stats)

    mesh = plsc.VectorSubcoreMesh(
        core_axis_name="c", subcore_axis_name="s", num_cores=_NC, num_subcores=_NS
    )
    sc_hist = functools.partial(
        pl.kernel,
        out_type=jax.ShapeDtypeStruct((2 * B, _HWORDS), jnp.float32),
        mesh=mesh,
        compiler_params=pltpu.CompilerParams(needs_layout_passes=False),
        scratch_types=[
            pltpu.VMEM((_SC_CHUNK,), jnp.int32),
            pltpu.VMEM((_SC_CHUNK,), jnp.int32),
            pltpu.VMEM((_HWORDS,), jnp.float32),
            pltpu.SemaphoreType.DMA,
            pltpu.SemaphoreType.DMA,
        ],
    )(functools.partial(_sc_hist_body, N))
    hists = sc_hist(fidx)

    out = pl.pallas_call(
        _pass2_body,
        in_specs=[pl.BlockSpec((2 * B, _HWORDS), lambda: (0, 0))],
        out_specs=pl.BlockSpec(memory_space=pltpu.SMEM),
        out_shape=jax.ShapeDtypeStruct((1, 1), jnp.float32),
    )(hists)
    return out[0, 0]


# speedup vs baseline: 1.9598x; 1.9598x over previous
import functools

import jax
import jax.numpy as jnp
from jax import lax
from jax.experimental import pallas as pl
from jax.experimental.pallas import tpu as pltpu
from jax.experimental.pallas import tpu_sc as plsc

_BINS = 50
_EPS = 1e-08

_NC = 2
_NS = 16
_L = 16

_BINS_PAD = 64
_NCOPY = 8
_HWORDS = _BINS_PAD * _L * _NCOPY
_SC_CHUNK = 32768


def _pass1_body(nchunk, rblk, p_ref, t_ref, o_ref, s_ref, mm_ref):
    j = pl.program_id(0)
    vals = []
    for r in range(rblk):
        p0 = p_ref[:, 0, r, :]
        p1 = p_ref[:, 1, r, :]
        t0 = t_ref[:, 0, r, :]
        t1 = t_ref[:, 1, r, :]
        vp = jnp.maximum(jnp.sqrt(p0 * p0 + p1 * p1), 1e-6)
        vt = jnp.maximum(jnp.sqrt(t0 * t0 + t1 * t1), 1e-6)
        w = p0.shape[-1]
        o_ref[0:16, r * w : (r + 1) * w] = vp
        o_ref[16:32, r * w : (r + 1) * w] = vt
        vals.append(vp)
        vals.append(vt)
    bmin_a = functools.reduce(jnp.minimum, vals)
    bmax_a = functools.reduce(jnp.maximum, vals)
    bmin = jnp.min(bmin_a)
    bmax = jnp.max(bmax_a)

    @pl.when(j == 0)
    def _():
        mm_ref[0] = bmin
        mm_ref[1] = bmax

    @pl.when(j > 0)
    def _():
        mm_ref[0] = jnp.minimum(mm_ref[0], bmin)
        mm_ref[1] = jnp.maximum(mm_ref[1], bmax)

    @pl.when(j == nchunk - 1)
    def _():
        mn = mm_ref[0]
        mx = mm_ref[1]
        scale = _BINS / jnp.maximum(mx - mn, 1e-12)
        s_ref[0, 0] = mn
        s_ref[0, 1] = scale


def _binpass_body(y_ref, s_ref, o_ref):
    mn = s_ref[0, 0]
    sc = s_ref[0, 1]
    y = y_ref[...]
    t = (y - mn) * sc
    t = jnp.maximum(jnp.minimum(t, float(_BINS - 1)), 0.0)
    idx = t.astype(jnp.int32)
    ci = lax.broadcasted_iota(jnp.int32, y.shape, 1)
    pat = (ci & (_L - 1)) + ((ci >> 4) & (_NCOPY - 1)) * (_BINS_PAD * _L)
    o_ref[...] = idx * _L + pat


def _sc_hist_body(n, fidx, out, buf0, buf1, hist, sem0, sem1):
    c = lax.axis_index("c")
    s = lax.axis_index("s")
    wid = c * _NS + s

    zv = jnp.zeros((_L,), jnp.float32)

    @plsc.parallel_loop(0, _HWORDS // _L, unroll=4)
    def _zero(i):
        hist[pl.ds(i * _L, _L)] = zv

    ones = jnp.ones((_L,), jnp.float32)
    nchunk = n // _SC_CHUNK
    nvec = _SC_CHUNK // _L
    bufs = (buf0, buf1)
    sems = (sem0, sem1)
    copies = [None, None]
    copies[0] = pltpu.async_copy(fidx.at[wid, pl.ds(0, _SC_CHUNK)], buf0, sem0)
    for k in range(nchunk):
        cur = k % 2
        if k + 1 < nchunk:
            copies[1 - cur] = pltpu.async_copy(
                fidx.at[wid, pl.ds((k + 1) * _SC_CHUNK, _SC_CHUNK)],
                bufs[1 - cur],
                sems[1 - cur],
            )
        copies[cur].wait()
        buf = bufs[cur]

        @plsc.parallel_loop(0, nvec, unroll=8)
        def body(i):
            fl = buf[pl.ds(i * _L, _L)]
            plsc.addupdate_scatter(hist, [fl], ones)

    pltpu.sync_copy(hist, out.at[wid])


def _pass2_body(h_ref, o_ref):
    xs = h_ref[...]
    x = xs[:, 0 : _BINS_PAD * _L]
    for j in range(1, _NCOPY):
        x = x + xs[:, j * _BINS_PAD * _L : (j + 1) * _BINS_PAD * _L]
    rows = lax.broadcasted_iota(jnp.int32, (_BINS_PAD * _L, 128), 0)
    cols = lax.broadcasted_iota(jnp.int32, (_BINS_PAD * _L, 128), 1)
    m = (rows // _L == cols).astype(jnp.float32)
    h = jnp.dot(x, m, preferred_element_type=jnp.float32)
    lanes = lax.broadcasted_iota(jnp.int32, (16, 128), 1)
    valid = lanes < _BINS
    hp = jnp.where(valid, h[0:16, :] + _EPS, 0.0)
    ht = jnp.where(valid, h[16:32, :] + _EPS, 0.0)
    hp = hp / jnp.sum(hp, axis=1, keepdims=True)
    ht = ht / jnp.sum(ht, axis=1, keepdims=True)
    kl = jnp.where(valid, ht * jnp.log(ht / hp), 0.0)
    o_ref[0, 0] = jnp.sum(kl) / 16.0


def kernel(y_pred, y_true):
    B, C, H, W = y_pred.shape
    N = H * W

    ch = 4096
    rblk = ch // W
    nchunk = N // ch
    yext, stats = pl.pallas_call(
        functools.partial(_pass1_body, nchunk, rblk),
        grid=(nchunk,),
        in_specs=[
            pl.BlockSpec((B, C, rblk, W), lambda j: (0, 0, j, 0)),
            pl.BlockSpec((B, C, rblk, W), lambda j: (0, 0, j, 0)),
        ],
        out_specs=[
            pl.BlockSpec((2 * B, ch), lambda j: (0, j)),
            pl.BlockSpec((1, 2), lambda j: (0, 0), memory_space=pltpu.SMEM),
        ],
        out_shape=[
            jax.ShapeDtypeStruct((2 * B, N), jnp.float32),
            jax.ShapeDtypeStruct((1, 2), jnp.float32),
        ],
        scratch_shapes=[pltpu.SMEM((2,), jnp.float32)],
    )(y_pred, y_true)

    fidx = pl.pallas_call(
        _binpass_body,
        grid=(nchunk,),
        in_specs=[
            pl.BlockSpec((2 * B, ch), lambda j: (0, j)),
            pl.BlockSpec((1, 2), lambda j: (0, 0), memory_space=pltpu.SMEM),
        ],
        out_specs=pl.BlockSpec((2 * B, ch), lambda j: (0, j)),
        out_shape=jax.ShapeDtypeStruct((2 * B, N), jnp.int32),
    )(yext, stats)

    mesh = plsc.VectorSubcoreMesh(
        core_axis_name="c", subcore_axis_name="s", num_cores=_NC, num_subcores=_NS
    )
    sc_hist = functools.partial(
        pl.kernel,
        out_type=jax.ShapeDtypeStruct((2 * B, _HWORDS), jnp.float32),
        mesh=mesh,
        compiler_params=pltpu.CompilerParams(needs_layout_passes=False),
        scratch_types=[
            pltpu.VMEM((_SC_CHUNK,), jnp.int32),
            pltpu.VMEM((_SC_CHUNK,), jnp.int32),
            pltpu.VMEM((_HWORDS,), jnp.float32),
            pltpu.SemaphoreType.DMA,
            pltpu.SemaphoreType.DMA,
        ],
    )(functools.partial(_sc_hist_body, N))
    hists = sc_hist(fidx)

    out = pl.pallas_call(
        _pass2_body,
        in_specs=[pl.BlockSpec((2 * B, _HWORDS), lambda: (0, 0))],
        out_specs=pl.BlockSpec(memory_space=pltpu.SMEM),
        out_shape=jax.ShapeDtypeStruct((1, 1), jnp.float32),
    )(hists)
    return out[0, 0]
